# baseline (speedup 1.0000x reference)
import jax
import jax.numpy as jnp
from jax import lax
from jax.experimental import pallas as pl
from jax.experimental.pallas import tpu as pltpu
from jax.experimental.pallas import tpu_sc as plsc

R = 32
N = 1_000_000
CH = 50_000
NCH = N // CH
BLK = 2_000
NBLK = CH // BLK
LANES = 16
NACC = 25


def _tree_sum(vs):
    vs = list(vs)
    while len(vs) > 1:
        nxt = [a + b for a, b in zip(vs[::2], vs[1::2])]
        if len(vs) % 2:
            nxt.append(vs[-1])
        vs = nxt
    return vs[0]


def _region_sum(buf, base, nvregs):
    iters = nvregs // NACC
    def body(i, acc):
        off = base + i * (NACC * LANES)
        vs = [buf[pl.ds(off + j * LANES, LANES)] for j in range(NACC)]
        return acc + _tree_sum(vs)
    acc = lax.fori_loop(0, iters, body, jnp.zeros((LANES,), jnp.float32))
    return jnp.sum(acc)


def _fine_count(buf, start, prefix, tv):
    def body(i, carry):
        run, c1, c2 = carry
        v = buf[pl.ds(start + i * LANES, LANES)]
        absc = plsc.cumsum(v) + jnp.full((LANES,), run)
        m = absc < tv
        c1 = c1 + m.astype(jnp.int32)
        c2 = c2 + m.astype(jnp.int32)
        return run + jnp.sum(v), c1, c2
    z = jnp.zeros((LANES,), jnp.int32)
    _, c1, c2 = lax.fori_loop(0, BLK // LANES, body, (prefix, z, z))
    return jnp.sum(c1), jnp.sum(c2)


def _scalar_scan(sums, t):
    run = jnp.float32(0.0)
    nbelow = jnp.int32(0)
    pfx = jnp.float32(0.0)
    for s in sums:
        run = run + s
        below = run < t
        nbelow = nbelow + below.astype(jnp.int32)
        pfx = pfx + jnp.where(below, s, jnp.float32(0.0))
    return nbelow, pfx


def _sc_body(x_hbm, u_hbm, out_hbm, bufa, bufb, u_v, flag_v, sema, semb):
    wid = lax.axis_index("s") * 2 + lax.axis_index("c")
    row = wid * N
    pltpu.sync_copy(u_hbm.at[pl.ds(wid * LANES, LANES)], u_v)

    bufs = (bufa, bufb)
    sems = (sema, semb)

    h = [None] * NCH
    h[0] = pltpu.async_copy(x_hbm.at[pl.ds(row, CH)], bufs[0], sems[0])
    chunk_sums = []
    for c in range(NCH):
        if c + 1 < NCH:
            h[c + 1] = pltpu.async_copy(
                x_hbm.at[pl.ds(row + (c + 1) * CH, CH)],
                bufs[(c + 1) % 2], sems[(c + 1) % 2])
        h[c].wait()
        chunk_sums.append(_region_sum(bufs[c % 2], 0, CH // LANES))
    total = _tree_sum(chunk_sums)

    u_s = u_v[...][0]
    t = u_s * total
    tv = jnp.full((LANES,), t)

    nfull, pfx = _scalar_scan(chunk_sums, t)
    c_star = jnp.minimum(nfull, NCH - 1)

    pltpu.sync_copy(x_hbm.at[pl.ds(row + c_star * CH, CH)], bufs[0])
    block_sums = [_region_sum(bufs[0], b * BLK, BLK // LANES)
                  for b in range(NBLK)]
    nb, bpfx = _scalar_scan(block_sums, t - pfx)
    b_star = jnp.minimum(nb, NBLK - 1)
    pfx2 = pfx + bpfx

    cnt1, cnt2 = _fine_count(bufs[0], b_star * BLK, pfx2, tv)
    idx1 = c_star * CH + b_star * BLK + cnt1
    idx2 = c_star * CH + b_star * BLK + cnt2

    neq = idx1 != idx2
    flag_v[...] = jnp.full((LANES,), jnp.where(neq, 1.0, 0.0)
                           .astype(jnp.float32))
    pltpu.sync_copy(flag_v, out_hbm.at[pl.ds(wid * LANES, LANES)])


def kernel(x):
    u = jax.random.uniform(jax.random.key(42), (R, 1), dtype=jnp.float32)
    ub = jnp.broadcast_to(u, (R, LANES)).reshape(R * LANES)
    mesh = plsc.VectorSubcoreMesh(core_axis_name="c", subcore_axis_name="s",
                                  num_cores=2, num_subcores=16)
    run = pl.kernel(
        _sc_body,
        out_type=jax.ShapeDtypeStruct((R * LANES,), jnp.float32),
        mesh=mesh,
        scratch_types=[
            pltpu.VMEM((CH,), jnp.float32),
            pltpu.VMEM((CH,), jnp.float32),
            pltpu.VMEM((LANES,), jnp.float32),
            pltpu.VMEM((LANES,), jnp.float32),
            pltpu.SemaphoreType.DMA,
            pltpu.SemaphoreType.DMA,
        ],
        compiler_params=pltpu.CompilerParams(needs_layout_passes=False),
    )
    flags = run(x.reshape(R * N), ub)
    return jnp.any(flags != 0.0).astype(jnp.float32)


# speedup vs baseline: 8.9696x; 8.9696x over previous
import jax
import jax.numpy as jnp
from jax import lax
from jax.experimental import pallas as pl
from jax.experimental.pallas import tpu as pltpu
from jax.experimental.pallas import tpu_sc as plsc

R = 32
N = 1_000_000
TCB = 2_048
NB = 512
LASTB = (N + TCB - 1) // TCB - 1
TPB = TCB // 128
NTILES = (N + 127) // 128
LANES = 16


def _tree_sum(vs):
    vs = list(vs)
    while len(vs) > 1:
        nxt = [a + b for a, b in zip(vs[::2], vs[1::2])]
        if len(vs) % 2:
            nxt.append(vs[-1])
        vs = nxt
    return vs[0]


def _tc_body(x_ref, o_ref):
    g = pl.program_id(0)
    i = pl.program_id(1)
    b = g * 128 + i
    col0 = b * TCB
    cols = col0 + lax.broadcasted_iota(jnp.int32, (R, TCB), 1)
    xb = jnp.where(cols < N, x_ref[...], jnp.float32(0.0))
    sums = jnp.sum(xb, axis=1, keepdims=True)
    lane = lax.broadcasted_iota(jnp.int32, (R, 128), 1)
    oh = jnp.where(lane == i, sums, jnp.float32(0.0))

    @pl.when(i == 0)
    def _init():
        o_ref[...] = oh

    @pl.when(i != 0)
    def _acc():
        o_ref[...] = o_ref[...] + oh


def _block_sums(x):
    return pl.pallas_call(
        _tc_body,
        grid=(NB // 128, 128),
        in_specs=[pl.BlockSpec(
            (R, TCB), lambda g, i: (0, jnp.minimum(g * 128 + i, LASTB)))],
        out_specs=pl.BlockSpec((R, 128), lambda g, i: (0, g)),
        out_shape=jax.ShapeDtypeStruct((R, NB), jnp.float32),
    )(x)


def _sc_body(x_hbm, b_hbm, u_hbm, out_hbm, bv, tbuf, u_v, flag_v, semf):
    wid = lax.axis_index("s") * 2 + lax.axis_index("c")
    rr = wid % 8
    rg8 = pl.multiple_of(wid - rr, 8)
    pltpu.sync_copy(u_hbm.at[pl.ds(wid * LANES, LANES)], u_v)
    pltpu.sync_copy(b_hbm.at[pl.ds(wid * NB, NB)], bv)

    vregs = [bv[pl.ds(i * LANES, LANES)] for i in range(NB // LANES)]
    total = jnp.sum(_tree_sum(vregs))
    u_s = u_v[...][0]
    t = u_s * total
    tv = jnp.full((LANES,), t)

    run = jnp.float32(0.0)
    nbv = jnp.zeros((LANES,), jnp.int32)
    pv = jnp.zeros((LANES,), jnp.float32)
    for i in range(NB // LANES):
        v = vregs[i]
        c = plsc.cumsum(v) + jnp.full((LANES,), run)
        m = c < tv
        nbv = nbv + m.astype(jnp.int32)
        pv = pv + jnp.where(m, v, jnp.float32(0.0))
        run = run + jnp.sum(v)
    b_star = jnp.minimum(jnp.sum(nbv), LASTB)
    prefix = jnp.sum(pv)

    base_tc = b_star * TPB
    hs = []
    for k in range(TPB):
        tc = jnp.minimum(base_tc + k, NTILES - 1)
        cb = pl.multiple_of(tc * 128, 128)
        hs.append(pltpu.async_copy(
            x_hbm.at[pl.ds(rg8, 8), pl.ds(cb, 128)], tbuf.at[k], semf))
    for h in hs:
        h.wait()

    iota = lax.iota(jnp.int32, LANES)
    run2 = prefix
    cnt1 = jnp.zeros((LANES,), jnp.int32)
    cnt2 = jnp.zeros((LANES,), jnp.int32)
    for k in range(TPB):
        real = jnp.full((LANES,), base_tc + k < NTILES)
        colbase = jnp.minimum(base_tc + k, NTILES - 1) * 128
        for j in range(8):
            v = tbuf[k, rr, pl.ds(j * LANES, LANES)]
            valid = ((colbase + j * LANES + iota) < N) & real
            vm = jnp.where(valid, v, jnp.float32(0.0))
            absc = plsc.cumsum(vm) + jnp.full((LANES,), run2)
            m = (absc < tv) & valid
            cnt1 = cnt1 + m.astype(jnp.int32)
            cnt2 = cnt2 + m.astype(jnp.int32)
            run2 = run2 + jnp.sum(vm)

    idx1 = b_star * TCB + jnp.sum(cnt1)
    idx2 = b_star * TCB + jnp.sum(cnt2)
    neq = idx1 != idx2
    flag_v[...] = jnp.full((LANES,), jnp.where(neq, 1.0, 0.0)
                           .astype(jnp.float32))
    pltpu.sync_copy(flag_v, out_hbm.at[pl.ds(wid * LANES, LANES)])


def kernel(x):
    u = jax.random.uniform(jax.random.key(42), (R, 1), dtype=jnp.float32)
    ub = jnp.broadcast_to(u, (R, LANES)).reshape(R * LANES)
    bsum = _block_sums(x).reshape(R * NB)
    mesh = plsc.VectorSubcoreMesh(core_axis_name="c", subcore_axis_name="s",
                                  num_cores=2, num_subcores=16)
    run = pl.kernel(
        _sc_body,
        out_type=jax.ShapeDtypeStruct((R * LANES,), jnp.float32),
        mesh=mesh,
        scratch_types=[
            pltpu.VMEM((NB,), jnp.float32),
            pltpu.VMEM((TPB, 8, 128), jnp.float32),
            pltpu.VMEM((LANES,), jnp.float32),
            pltpu.VMEM((LANES,), jnp.float32),
            pltpu.SemaphoreType.DMA,
        ],
        compiler_params=pltpu.CompilerParams(needs_layout_passes=False),
    )
    flags = run(x, bsum, ub)
    return jnp.any(flags != 0.0).astype(jnp.float32)


# speedup vs baseline: 21.8492x; 2.4359x over previous
import jax
import jax.numpy as jnp
from jax import lax
from jax.experimental import pallas as pl
from jax.experimental.pallas import tpu as pltpu
from jax.experimental.pallas import tpu_sc as plsc

R = 32
N = 1_000_000
TCB = 8_192
NB = 128
LASTB = (N + TCB - 1) // TCB - 1
TPB = TCB // 128
NTILES = (N + 127) // 128
LANES = 16
DMA_ROUND = 16


def _tree_sum(vs):
    vs = list(vs)
    while len(vs) > 1:
        nxt = [a + b for a, b in zip(vs[::2], vs[1::2])]
        if len(vs) % 2:
            nxt.append(vs[-1])
        vs = nxt
    return vs[0]


def _tc_body(x_ref, o_ref):
    b = pl.program_id(0)
    lane = lax.broadcasted_iota(jnp.int32, (R, NB), 1)

    @pl.when(b == 0)
    def _init():
        o_ref[...] = jnp.zeros((R, NB), jnp.float32)

    @pl.when(b < LASTB)
    def _full():
        sums = jnp.sum(x_ref[...], axis=1, keepdims=True)
        o_ref[...] = o_ref[...] + jnp.where(lane == b, sums, jnp.float32(0.0))

    @pl.when(b >= LASTB)
    def _tail():
        cols = b * TCB + lax.broadcasted_iota(jnp.int32, (R, TCB), 1)
        xb = jnp.where(cols < N, x_ref[...], jnp.float32(0.0))
        sums = jnp.sum(xb, axis=1, keepdims=True)
        o_ref[...] = o_ref[...] + jnp.where(lane == b, sums, jnp.float32(0.0))


def _block_sums(x):
    return pl.pallas_call(
        _tc_body,
        grid=(NB,),
        in_specs=[pl.BlockSpec(
            (R, TCB), lambda b: (0, jnp.minimum(b, LASTB)))],
        out_specs=pl.BlockSpec((R, NB), lambda b: (0, 0)),
        out_shape=jax.ShapeDtypeStruct((R, NB), jnp.float32),
    )(x)


def _sc_body(x_hbm, b_hbm, u_hbm, out_hbm, bv, tbuf, u_v, flag_v, semf):
    wid = lax.axis_index("s") * 2 + lax.axis_index("c")
    rr = wid % 8
    rg8 = pl.multiple_of(wid - rr, 8)
    pltpu.sync_copy(u_hbm.at[pl.ds(wid * LANES, LANES)], u_v)
    pltpu.sync_copy(b_hbm.at[pl.ds(wid * NB, NB)], bv)

    vregs = [bv[pl.ds(i * LANES, LANES)] for i in range(NB // LANES)]
    total = jnp.sum(_tree_sum(vregs))
    u_s = u_v[...][0]
    t = u_s * total
    tv = jnp.full((LANES,), t)

    run = jnp.float32(0.0)
    nbv = jnp.zeros((LANES,), jnp.int32)
    pv = jnp.zeros((LANES,), jnp.float32)
    for i in range(NB // LANES):
        v = vregs[i]
        c = plsc.cumsum(v) + jnp.full((LANES,), run)
        m = c < tv
        nbv = nbv + m.astype(jnp.int32)
        pv = pv + jnp.where(m, v, jnp.float32(0.0))
        run = run + jnp.sum(v)
    b_star = jnp.minimum(jnp.sum(nbv), LASTB)
    prefix = jnp.sum(pv)

    base_tc = b_star * TPB
    iota = lax.iota(jnp.int32, LANES)
    for k0 in range(0, TPB, DMA_ROUND):
        hs = []
        for k in range(k0, k0 + DMA_ROUND):
            tc = jnp.minimum(base_tc + k, NTILES - 1)
            cb = pl.multiple_of(tc * 128, 128)
            hs.append(pltpu.async_copy(
                x_hbm.at[pl.ds(rg8, 8), pl.ds(cb, 128)], tbuf.at[k], semf))
        for h in hs:
            h.wait()

    tile_sums = []
    for k in range(TPB):
        real = base_tc + k < NTILES
        colbase = jnp.minimum(base_tc + k, NTILES - 1) * 128
        parts = []
        for j in range(8):
            v = tbuf[k, rr, pl.ds(j * LANES, LANES)]
            valid = ((colbase + j * LANES + iota) < N) & jnp.full(
                (LANES,), real)
            parts.append(jnp.where(valid, v, jnp.float32(0.0)))
        tile_sums.append(jnp.sum(_tree_sum(parts)))

    run2 = prefix
    ntile = jnp.int32(0)
    pfx2 = prefix
    for s in tile_sums:
        run2 = run2 + s
        below = run2 < t
        ntile = ntile + below.astype(jnp.int32)
        pfx2 = pfx2 + jnp.where(below, s, jnp.float32(0.0))
    k_star = jnp.minimum(ntile, TPB - 1)

    kcol = jnp.minimum(base_tc + k_star, NTILES - 1) * 128
    kreal = jnp.full((LANES,), base_tc + k_star < NTILES)
    run3 = pfx2
    cnt1 = jnp.zeros((LANES,), jnp.int32)
    cnt2 = jnp.zeros((LANES,), jnp.int32)
    for j in range(8):
        v = tbuf[k_star, rr, pl.ds(j * LANES, LANES)]
        valid = ((kcol + j * LANES + iota) < N) & kreal
        vm = jnp.where(valid, v, jnp.float32(0.0))
        absc = plsc.cumsum(vm) + jnp.full((LANES,), run3)
        m = (absc < tv) & valid
        cnt1 = cnt1 + m.astype(jnp.int32)
        cnt2 = cnt2 + m.astype(jnp.int32)
        run3 = run3 + jnp.sum(vm)

    idx1 = b_star * TCB + k_star * 128 + jnp.sum(cnt1)
    idx2 = b_star * TCB + k_star * 128 + jnp.sum(cnt2)
    neq = idx1 != idx2
    flag_v[...] = jnp.full((LANES,), jnp.where(neq, 1.0, 0.0)
                           .astype(jnp.float32))
    pltpu.sync_copy(flag_v, out_hbm.at[pl.ds(wid * LANES, LANES)])


def kernel(x):
    u = jax.random.uniform(jax.random.key(42), (R, 1), dtype=jnp.float32)
    ub = jnp.broadcast_to(u, (R, LANES)).reshape(R * LANES)
    bsum = _block_sums(x).reshape(R * NB)
    mesh = plsc.VectorSubcoreMesh(core_axis_name="c", subcore_axis_name="s",
                                  num_cores=2, num_subcores=16)
    run = pl.kernel(
        _sc_body,
        out_type=jax.ShapeDtypeStruct((R * LANES,), jnp.float32),
        mesh=mesh,
        scratch_types=[
            pltpu.VMEM((NB,), jnp.float32),
            pltpu.VMEM((TPB, 8, 128), jnp.float32),
            pltpu.VMEM((LANES,), jnp.float32),
            pltpu.VMEM((LANES,), jnp.float32),
            pltpu.SemaphoreType.DMA,
        ],
        compiler_params=pltpu.CompilerParams(needs_layout_passes=False),
    )
    flags = run(x, bsum, ub)
    return jnp.any(flags != 0.0).astype(jnp.float32)


# speedup vs baseline: 32.1067x; 1.4695x over previous
import jax
import jax.numpy as jnp
from jax import lax
from jax.experimental import pallas as pl
from jax.experimental.pallas import tpu as pltpu
from jax.experimental.pallas import tpu_sc as plsc

R = 32
N = 1_000_000
STEP = 16_384
NSUB = 8
FB = STEP // NSUB
NB = 512
LASTB = (N + FB - 1) // FB - 1
TCSTEPS = (N + STEP - 1) // STEP
TPB = FB // 128
NTILES = (N + 127) // 128
LANES = 16
DMA_ROUND = 16


def _tree_sum(vs):
    vs = list(vs)
    while len(vs) > 1:
        nxt = [a + b for a, b in zip(vs[::2], vs[1::2])]
        if len(vs) % 2:
            nxt.append(vs[-1])
        vs = nxt
    return vs[0]


def _tc_body(x_ref, o_ref):
    b = pl.program_id(0)
    lane = lax.broadcasted_iota(jnp.int32, (R, NB), 1)

    @pl.when(b == 0)
    def _init():
        o_ref[...] = jnp.zeros((R, NB), jnp.float32)

    @pl.when(b < TCSTEPS - 1)
    def _full():
        acc = jnp.zeros((R, NB), jnp.float32)
        for q in range(NSUB):
            sums = jnp.sum(x_ref[:, q * FB:(q + 1) * FB], axis=1,
                           keepdims=True)
            acc = acc + jnp.where(lane == b * NSUB + q, sums,
                                  jnp.float32(0.0))
        o_ref[...] = o_ref[...] + acc

    @pl.when(b == TCSTEPS - 1)
    def _tail():
        cols = b * STEP + lax.broadcasted_iota(jnp.int32, (R, STEP), 1)
        xb = jnp.where(cols < N, x_ref[...], jnp.float32(0.0))
        acc = jnp.zeros((R, NB), jnp.float32)
        for q in range(NSUB):
            sums = jnp.sum(xb[:, q * FB:(q + 1) * FB], axis=1,
                           keepdims=True)
            acc = acc + jnp.where(lane == b * NSUB + q, sums,
                                  jnp.float32(0.0))
        o_ref[...] = o_ref[...] + acc


def _block_sums(x):
    return pl.pallas_call(
        _tc_body,
        grid=(TCSTEPS,),
        in_specs=[pl.BlockSpec(
            (R, STEP), lambda b: (0, jnp.minimum(b, TCSTEPS - 1)))],
        out_specs=pl.BlockSpec((R, NB), lambda b: (0, 0)),
        out_shape=jax.ShapeDtypeStruct((R, NB), jnp.float32),
    )(x)


def _sc_body(x_hbm, b_hbm, u_hbm, out_hbm, bv, tbuf, u_v, flag_v, semf):
    wid = lax.axis_index("s") * 2 + lax.axis_index("c")
    rr = wid % 8
    rg8 = pl.multiple_of(wid - rr, 8)
    pltpu.sync_copy(u_hbm.at[pl.ds(wid * LANES, LANES)], u_v)
    pltpu.sync_copy(b_hbm.at[pl.ds(wid * NB, NB)], bv)

    vregs = [bv[pl.ds(i * LANES, LANES)] for i in range(NB // LANES)]
    total = jnp.sum(_tree_sum(vregs))
    u_s = u_v[...][0]
    t = u_s * total
    tv = jnp.full((LANES,), t)

    run = jnp.float32(0.0)
    nbv = jnp.zeros((LANES,), jnp.int32)
    pv = jnp.zeros((LANES,), jnp.float32)
    for i in range(NB // LANES):
        v = vregs[i]
        c = plsc.cumsum(v) + jnp.full((LANES,), run)
        m = c < tv
        nbv = nbv + m.astype(jnp.int32)
        pv = pv + jnp.where(m, v, jnp.float32(0.0))
        run = run + jnp.sum(v)
    b_star = jnp.minimum(jnp.sum(nbv), LASTB)
    prefix = jnp.sum(pv)

    base_tc = b_star * TPB
    iota = lax.iota(jnp.int32, LANES)
    for k0 in range(0, TPB, DMA_ROUND):
        hs = []
        for k in range(k0, k0 + DMA_ROUND):
            tc = jnp.minimum(base_tc + k, NTILES - 1)
            cb = pl.multiple_of(tc * 128, 128)
            hs.append(pltpu.async_copy(
                x_hbm.at[pl.ds(rg8, 8), pl.ds(cb, 128)], tbuf.at[k], semf))
        for h in hs:
            h.wait()

    tile_sums = []
    for k in range(TPB):
        real = base_tc + k < NTILES
        colbase = jnp.minimum(base_tc + k, NTILES - 1) * 128
        parts = []
        for j in range(8):
            v = tbuf[k, rr, pl.ds(j * LANES, LANES)]
            valid = ((colbase + j * LANES + iota) < N) & jnp.full(
                (LANES,), real)
            parts.append(jnp.where(valid, v, jnp.float32(0.0)))
        tile_sums.append(jnp.sum(_tree_sum(parts)))

    run2 = prefix
    ntile = jnp.int32(0)
    pfx2 = prefix
    for s in tile_sums:
        run2 = run2 + s
        below = run2 < t
        ntile = ntile + below.astype(jnp.int32)
        pfx2 = pfx2 + jnp.where(below, s, jnp.float32(0.0))
    k_star = jnp.minimum(ntile, TPB - 1)

    kcol = jnp.minimum(base_tc + k_star, NTILES - 1) * 128
    kreal = jnp.full((LANES,), base_tc + k_star < NTILES)
    run3 = pfx2
    cnt1 = jnp.zeros((LANES,), jnp.int32)
    cnt2 = jnp.zeros((LANES,), jnp.int32)
    for j in range(8):
        v = tbuf[k_star, rr, pl.ds(j * LANES, LANES)]
        valid = ((kcol + j * LANES + iota) < N) & kreal
        vm = jnp.where(valid, v, jnp.float32(0.0))
        absc = plsc.cumsum(vm) + jnp.full((LANES,), run3)
        m = (absc < tv) & valid
        cnt1 = cnt1 + m.astype(jnp.int32)
        cnt2 = cnt2 + m.astype(jnp.int32)
        run3 = run3 + jnp.sum(vm)

    idx1 = b_star * FB + k_star * 128 + jnp.sum(cnt1)
    idx2 = b_star * FB + k_star * 128 + jnp.sum(cnt2)
    neq = idx1 != idx2
    flag_v[...] = jnp.full((LANES,), jnp.where(neq, 1.0, 0.0)
                           .astype(jnp.float32))
    pltpu.sync_copy(flag_v, out_hbm.at[pl.ds(wid * LANES, LANES)])


def kernel(x):
    u = jax.random.uniform(jax.random.key(42), (R, 1), dtype=jnp.float32)
    ub = jnp.broadcast_to(u, (R, LANES)).reshape(R * LANES)
    bsum = _block_sums(x).reshape(R * NB)
    mesh = plsc.VectorSubcoreMesh(core_axis_name="c", subcore_axis_name="s",
                                  num_cores=2, num_subcores=16)
    run = pl.kernel(
        _sc_body,
        out_type=jax.ShapeDtypeStruct((R * LANES,), jnp.float32),
        mesh=mesh,
        scratch_types=[
            pltpu.VMEM((NB,), jnp.float32),
            pltpu.VMEM((TPB, 8, 128), jnp.float32),
            pltpu.VMEM((LANES,), jnp.float32),
            pltpu.VMEM((LANES,), jnp.float32),
            pltpu.SemaphoreType.DMA,
        ],
        compiler_params=pltpu.CompilerParams(needs_layout_passes=False),
    )
    flags = run(x, bsum, ub)
    return jnp.any(flags != 0.0).astype(jnp.float32)


# speedup vs baseline: 39.1933x; 1.2207x over previous
import jax
import jax.numpy as jnp
from jax import lax
from jax.experimental import pallas as pl
from jax.experimental.pallas import tpu as pltpu
from jax.experimental.pallas import tpu_sc as plsc

R = 32
N = 1_000_000
STEP = 32_768
NSUB = 16
FB = STEP // NSUB
NB = 512
LASTB = (N + FB - 1) // FB - 1
TCSTEPS = (N + STEP - 1) // STEP
TPB = FB // 128
NTILES = (N + 127) // 128
LANES = 16
DMA_ROUND = 16


def _tree_sum(vs):
    vs = list(vs)
    while len(vs) > 1:
        nxt = [a + b for a, b in zip(vs[::2], vs[1::2])]
        if len(vs) % 2:
            nxt.append(vs[-1])
        vs = nxt
    return vs[0]


def _tc_body(x_ref, o_ref):
    b = pl.program_id(0)
    lane = lax.broadcasted_iota(jnp.int32, (R, NB), 1)

    @pl.when(b == 0)
    def _init():
        o_ref[...] = jnp.zeros((R, NB), jnp.float32)

    @pl.when(b < TCSTEPS - 1)
    def _full():
        acc = jnp.zeros((R, NB), jnp.float32)
        for q in range(NSUB):
            sums = jnp.sum(x_ref[:, q * FB:(q + 1) * FB], axis=1,
                           keepdims=True)
            acc = acc + jnp.where(lane == b * NSUB + q, sums,
                                  jnp.float32(0.0))
        o_ref[...] = o_ref[...] + acc

    @pl.when(b == TCSTEPS - 1)
    def _tail():
        cols = b * STEP + lax.broadcasted_iota(jnp.int32, (R, STEP), 1)
        xb = jnp.where(cols < N, x_ref[...], jnp.float32(0.0))
        acc = jnp.zeros((R, NB), jnp.float32)
        for q in range(NSUB):
            sums = jnp.sum(xb[:, q * FB:(q + 1) * FB], axis=1,
                           keepdims=True)
            acc = acc + jnp.where(lane == b * NSUB + q, sums,
                                  jnp.float32(0.0))
        o_ref[...] = o_ref[...] + acc


def _block_sums(x):
    return pl.pallas_call(
        _tc_body,
        grid=(TCSTEPS,),
        in_specs=[pl.BlockSpec(
            (R, STEP), lambda b: (0, jnp.minimum(b, TCSTEPS - 1)))],
        out_specs=pl.BlockSpec((R, NB), lambda b: (0, 0)),
        out_shape=jax.ShapeDtypeStruct((R, NB), jnp.float32),
    )(x)


def _sc_body(x_hbm, b_hbm, u_hbm, out_hbm, bv, tbuf, u_v, flag_v, semf):
    wid = lax.axis_index("s") * 2 + lax.axis_index("c")
    rr = wid % 8
    rg8 = pl.multiple_of(wid - rr, 8)
    pltpu.sync_copy(u_hbm.at[pl.ds(wid * LANES, LANES)], u_v)
    pltpu.sync_copy(b_hbm.at[pl.ds(wid * NB, NB)], bv)

    vregs = [bv[pl.ds(i * LANES, LANES)] for i in range(NB // LANES)]
    total = jnp.sum(_tree_sum(vregs))
    u_s = u_v[...][0]
    t = u_s * total
    tv = jnp.full((LANES,), t)

    run = jnp.float32(0.0)
    nbv = jnp.zeros((LANES,), jnp.int32)
    pv = jnp.zeros((LANES,), jnp.float32)
    for i in range(NB // LANES):
        v = vregs[i]
        c = plsc.cumsum(v) + jnp.full((LANES,), run)
        m = c < tv
        nbv = nbv + m.astype(jnp.int32)
        pv = pv + jnp.where(m, v, jnp.float32(0.0))
        run = run + jnp.sum(v)
    b_star = jnp.minimum(jnp.sum(nbv), LASTB)
    prefix = jnp.sum(pv)

    base_tc = b_star * TPB
    iota = lax.iota(jnp.int32, LANES)
    for k0 in range(0, TPB, DMA_ROUND):
        hs = []
        for k in range(k0, k0 + DMA_ROUND):
            tc = jnp.minimum(base_tc + k, NTILES - 1)
            cb = pl.multiple_of(tc * 128, 128)
            hs.append(pltpu.async_copy(
                x_hbm.at[pl.ds(rg8, 8), pl.ds(cb, 128)], tbuf.at[k], semf))
        for h in hs:
            h.wait()

    tile_sums = []
    for k in range(TPB):
        real = base_tc + k < NTILES
        colbase = jnp.minimum(base_tc + k, NTILES - 1) * 128
        parts = []
        for j in range(8):
            v = tbuf[k, rr, pl.ds(j * LANES, LANES)]
            valid = ((colbase + j * LANES + iota) < N) & jnp.full(
                (LANES,), real)
            parts.append(jnp.where(valid, v, jnp.float32(0.0)))
        tile_sums.append(jnp.sum(_tree_sum(parts)))

    run2 = prefix
    ntile = jnp.int32(0)
    pfx2 = prefix
    for s in tile_sums:
        run2 = run2 + s
        below = run2 < t
        ntile = ntile + below.astype(jnp.int32)
        pfx2 = pfx2 + jnp.where(below, s, jnp.float32(0.0))
    k_star = jnp.minimum(ntile, TPB - 1)

    kcol = jnp.minimum(base_tc + k_star, NTILES - 1) * 128
    kreal = jnp.full((LANES,), base_tc + k_star < NTILES)
    run3 = pfx2
    cnt1 = jnp.zeros((LANES,), jnp.int32)
    cnt2 = jnp.zeros((LANES,), jnp.int32)
    for j in range(8):
        v = tbuf[k_star, rr, pl.ds(j * LANES, LANES)]
        valid = ((kcol + j * LANES + iota) < N) & kreal
        vm = jnp.where(valid, v, jnp.float32(0.0))
        absc = plsc.cumsum(vm) + jnp.full((LANES,), run3)
        m = (absc < tv) & valid
        cnt1 = cnt1 + m.astype(jnp.int32)
        cnt2 = cnt2 + m.astype(jnp.int32)
        run3 = run3 + jnp.sum(vm)

    idx1 = b_star * FB + k_star * 128 + jnp.sum(cnt1)
    idx2 = b_star * FB + k_star * 128 + jnp.sum(cnt2)
    neq = idx1 != idx2
    flag_v[...] = jnp.full((LANES,), jnp.where(neq, 1.0, 0.0)
                           .astype(jnp.float32))
    pltpu.sync_copy(flag_v, out_hbm.at[pl.ds(wid * LANES, LANES)])


def kernel(x):
    u = jax.random.uniform(jax.random.key(42), (R, 1), dtype=jnp.float32)
    ub = jnp.broadcast_to(u, (R, LANES)).reshape(R * LANES)
    bsum = _block_sums(x).reshape(R * NB)
    mesh = plsc.VectorSubcoreMesh(core_axis_name="c", subcore_axis_name="s",
                                  num_cores=2, num_subcores=16)
    run = pl.kernel(
        _sc_body,
        out_type=jax.ShapeDtypeStruct((R * LANES,), jnp.float32),
        mesh=mesh,
        scratch_types=[
            pltpu.VMEM((NB,), jnp.float32),
            pltpu.VMEM((TPB, 8, 128), jnp.float32),
            pltpu.VMEM((LANES,), jnp.float32),
            pltpu.VMEM((LANES,), jnp.float32),
            pltpu.SemaphoreType.DMA,
        ],
        compiler_params=pltpu.CompilerParams(needs_layout_passes=False),
    )
    flags = run(x, bsum, ub)
    return jnp.any(flags != 0.0).astype(jnp.float32)


# speedup vs baseline: 42.4134x; 1.0822x over previous
import jax
import jax.numpy as jnp
from jax import lax
from jax.experimental import pallas as pl
from jax.experimental.pallas import tpu as pltpu
from jax.experimental.pallas import tpu_sc as plsc

R = 32
N = 1_000_000
STEP = 65_536
NSUB = 32
FB = STEP // NSUB
NB = 512
LASTB = (N + FB - 1) // FB - 1
TCSTEPS = (N + STEP - 1) // STEP
TPB = FB // 128
NTILES = (N + 127) // 128
LANES = 16
DMA_ROUND = 16


def _tree_sum(vs):
    vs = list(vs)
    while len(vs) > 1:
        nxt = [a + b for a, b in zip(vs[::2], vs[1::2])]
        if len(vs) % 2:
            nxt.append(vs[-1])
        vs = nxt
    return vs[0]


def _tc_body(x_ref, o_ref):
    b = pl.program_id(0)
    lane = lax.broadcasted_iota(jnp.int32, (R, NB), 1)

    @pl.when(b == 0)
    def _init():
        o_ref[...] = jnp.zeros((R, NB), jnp.float32)

    @pl.when(b < TCSTEPS - 1)
    def _full():
        acc = jnp.zeros((R, NB), jnp.float32)
        for q in range(NSUB):
            sums = jnp.sum(x_ref[:, q * FB:(q + 1) * FB], axis=1,
                           keepdims=True)
            acc = acc + jnp.where(lane == b * NSUB + q, sums,
                                  jnp.float32(0.0))
        o_ref[...] = o_ref[...] + acc

    @pl.when(b == TCSTEPS - 1)
    def _tail():
        cols = b * STEP + lax.broadcasted_iota(jnp.int32, (R, STEP), 1)
        xb = jnp.where(cols < N, x_ref[...], jnp.float32(0.0))
        acc = jnp.zeros((R, NB), jnp.float32)
        for q in range(NSUB):
            sums = jnp.sum(xb[:, q * FB:(q + 1) * FB], axis=1,
                           keepdims=True)
            acc = acc + jnp.where(lane == b * NSUB + q, sums,
                                  jnp.float32(0.0))
        o_ref[...] = o_ref[...] + acc


def _block_sums(x):
    return pl.pallas_call(
        _tc_body,
        grid=(TCSTEPS,),
        in_specs=[pl.BlockSpec(
            (R, STEP), lambda b: (0, jnp.minimum(b, TCSTEPS - 1)))],
        out_specs=pl.BlockSpec((R, NB), lambda b: (0, 0)),
        out_shape=jax.ShapeDtypeStruct((R, NB), jnp.float32),
    )(x)


def _sc_body(x_hbm, b_hbm, u_hbm, out_hbm, bv, tbuf, u_v, flag_v, semf):
    wid = lax.axis_index("s") * 2 + lax.axis_index("c")
    rr = wid % 8
    rg8 = pl.multiple_of(wid - rr, 8)
    pltpu.sync_copy(u_hbm.at[pl.ds(wid * LANES, LANES)], u_v)
    pltpu.sync_copy(b_hbm.at[pl.ds(wid * NB, NB)], bv)

    vregs = [bv[pl.ds(i * LANES, LANES)] for i in range(NB // LANES)]
    total = jnp.sum(_tree_sum(vregs))
    u_s = u_v[...][0]
    t = u_s * total
    tv = jnp.full((LANES,), t)

    run = jnp.float32(0.0)
    nbv = jnp.zeros((LANES,), jnp.int32)
    pv = jnp.zeros((LANES,), jnp.float32)
    for i in range(NB // LANES):
        v = vregs[i]
        c = plsc.cumsum(v) + jnp.full((LANES,), run)
        m = c < tv
        nbv = nbv + m.astype(jnp.int32)
        pv = pv + jnp.where(m, v, jnp.float32(0.0))
        run = run + jnp.sum(v)
    b_star = jnp.minimum(jnp.sum(nbv), LASTB)
    prefix = jnp.sum(pv)

    base_tc = b_star * TPB
    iota = lax.iota(jnp.int32, LANES)
    for k0 in range(0, TPB, DMA_ROUND):
        hs = []
        for k in range(k0, k0 + DMA_ROUND):
            tc = jnp.minimum(base_tc + k, NTILES - 1)
            cb = pl.multiple_of(tc * 128, 128)
            hs.append(pltpu.async_copy(
                x_hbm.at[pl.ds(rg8, 8), pl.ds(cb, 128)], tbuf.at[k], semf))
        for h in hs:
            h.wait()

    tile_sums = []
    for k in range(TPB):
        real = base_tc + k < NTILES
        colbase = jnp.minimum(base_tc + k, NTILES - 1) * 128
        parts = []
        for j in range(8):
            v = tbuf[k, rr, pl.ds(j * LANES, LANES)]
            valid = ((colbase + j * LANES + iota) < N) & jnp.full(
                (LANES,), real)
            parts.append(jnp.where(valid, v, jnp.float32(0.0)))
        tile_sums.append(jnp.sum(_tree_sum(parts)))

    run2 = prefix
    ntile = jnp.int32(0)
    pfx2 = prefix
    for s in tile_sums:
        run2 = run2 + s
        below = run2 < t
        ntile = ntile + below.astype(jnp.int32)
        pfx2 = pfx2 + jnp.where(below, s, jnp.float32(0.0))
    k_star = jnp.minimum(ntile, TPB - 1)

    kcol = jnp.minimum(base_tc + k_star, NTILES - 1) * 128
    kreal = jnp.full((LANES,), base_tc + k_star < NTILES)
    run3 = pfx2
    cnt1 = jnp.zeros((LANES,), jnp.int32)
    cnt2 = jnp.zeros((LANES,), jnp.int32)
    for j in range(8):
        v = tbuf[k_star, rr, pl.ds(j * LANES, LANES)]
        valid = ((kcol + j * LANES + iota) < N) & kreal
        vm = jnp.where(valid, v, jnp.float32(0.0))
        absc = plsc.cumsum(vm) + jnp.full((LANES,), run3)
        m = (absc < tv) & valid
        cnt1 = cnt1 + m.astype(jnp.int32)
        cnt2 = cnt2 + m.astype(jnp.int32)
        run3 = run3 + jnp.sum(vm)

    idx1 = b_star * FB + k_star * 128 + jnp.sum(cnt1)
    idx2 = b_star * FB + k_star * 128 + jnp.sum(cnt2)
    neq = idx1 != idx2
    flag_v[...] = jnp.full((LANES,), jnp.where(neq, 1.0, 0.0)
                           .astype(jnp.float32))
    pltpu.sync_copy(flag_v, out_hbm.at[pl.ds(wid * LANES, LANES)])


def kernel(x):
    u = jax.random.uniform(jax.random.key(42), (R, 1), dtype=jnp.float32)
    ub = jnp.broadcast_to(u, (R, LANES)).reshape(R * LANES)
    bsum = _block_sums(x).reshape(R * NB)
    mesh = plsc.VectorSubcoreMesh(core_axis_name="c", subcore_axis_name="s",
                                  num_cores=2, num_subcores=16)
    run = pl.kernel(
        _sc_body,
        out_type=jax.ShapeDtypeStruct((R * LANES,), jnp.float32),
        mesh=mesh,
        scratch_types=[
            pltpu.VMEM((NB,), jnp.float32),
            pltpu.VMEM((TPB, 8, 128), jnp.float32),
            pltpu.VMEM((LANES,), jnp.float32),
            pltpu.VMEM((LANES,), jnp.float32),
            pltpu.SemaphoreType.DMA,
        ],
        compiler_params=pltpu.CompilerParams(needs_layout_passes=False),
    )
    flags = run(x, bsum, ub)
    return jnp.any(flags != 0.0).astype(jnp.float32)
